# baseline (speedup 1.0000x reference)
import functools

import jax
import jax.numpy as jnp
from jax import lax
from jax.experimental import pallas as pl
from jax.experimental.pallas import tpu as pltpu
from jax.experimental.pallas import tpu_sc as plsc

_MIN_NORM = 1e-15
_BALL_EPS = 4e-3
_N, _E, _D = 10000, 320000, 128

_NC, _NS = 2, 16
_N_PAD = 10240
_ROWS_PER_TILE = _N_PAD // _NS
_CHUNK = 80
_E_PER_CORE = _E // _NC
_E_PER_TILE = _E_PER_CORE // _NS
_NPT = _E_PER_TILE // _CHUNK
_ZROWS = 80
_DEG_W = 16


def _rnorm(v):
    return jnp.maximum(jnp.sqrt(jnp.sum(v * v, axis=-1, keepdims=True)), _MIN_NORM)


def _artanh(v):
    v = jnp.clip(v, -1.0 + 1e-7, 1.0 - 1e-7)
    return 0.5 * (jnp.log1p(v) - jnp.log1p(-v))


def _proj(v):
    n = _rnorm(v)
    maxnorm = 1.0 - _BALL_EPS
    return jnp.where(n > maxnorm, v / n * maxnorm, v)


def _expmap0(v):
    n = _rnorm(v)
    return jnp.tanh(n) * v / n


def _logmap0(v):
    n = _rnorm(v)
    return v / n * _artanh(n)


def _phase_a_body(x_ref, w_ref, b_ref, o_ref):
    x = x_ref[...]
    w = w_ref[...]
    b = b_ref[...]

    x_hyp = _proj(_expmap0(x))

    xn = _rnorm(x_hyp)
    mx = lax.dot_general(x_hyp, w, (((1,), (1,)), ((), ())),
                         preferred_element_type=jnp.float32,
                         precision=lax.Precision.HIGHEST)
    mxn = _rnorm(mx)
    res = jnp.tanh(mxn / xn * _artanh(xn)) * mx / mxn
    res = jnp.where(jnp.all(mx == 0.0, axis=-1, keepdims=True), 0.0, res)
    res = _proj(res)

    hb = _proj(_expmap0(b))

    x2 = jnp.sum(res * res, axis=-1, keepdims=True)
    y2 = jnp.sum(hb * hb, axis=-1, keepdims=True)
    xy = jnp.sum(res * hb, axis=-1, keepdims=True)
    num = (1.0 + 2.0 * xy + y2) * res + (1.0 - x2) * hb
    den = jnp.maximum(1.0 + 2.0 * xy + x2 * y2, _MIN_NORM)
    ma = _proj(num / den)

    ht = _logmap0(ma)
    blk = ht.shape[0]
    hi = ht.reshape(blk, _D // 32, 2, 16)
    o_ref[...] = jnp.swapaxes(hi, 2, 3).reshape(blk, _D).astype(jnp.bfloat16)


def _phase_a(x, W, b2):
    blk = 1000
    return pl.pallas_call(
        _phase_a_body,
        grid=(_N // blk,),
        in_specs=[
            pl.BlockSpec((blk, _D), lambda i: (i, 0)),
            pl.BlockSpec((_D, _D), lambda i: (0, 0)),
            pl.BlockSpec((1, _D), lambda i: (0, 0)),
        ],
        out_specs=pl.BlockSpec((blk, _D), lambda i: (i, 0)),
        out_shape=jax.ShapeDtypeStruct((_N, _D), jnp.bfloat16),
    )(x, W, b2)


def _sc_agg(h_tan, edge_index):
    @functools.partial(
        pl.kernel,
        out_type=[
            jax.ShapeDtypeStruct((_NC, _N_PAD, _D), jnp.float32),
            jax.ShapeDtypeStruct((_NC, _N_PAD, _DEG_W), jnp.float32),
        ],
        mesh=plsc.VectorSubcoreMesh(core_axis_name="c", subcore_axis_name="s"),
        compiler_params=pltpu.CompilerParams(use_tc_tiling_on_sc=False,
                                             needs_layout_passes=False),
        scratch_types=[
            pltpu.VMEM_SHARED((_N_PAD, _D), jnp.float32),
            pltpu.VMEM_SHARED((_N_PAD, _DEG_W), jnp.float32),
            pltpu.VMEM((2, _CHUNK), jnp.int32),
            pltpu.VMEM((2, _CHUNK), jnp.int32),
            pltpu.VMEM((_CHUNK, _D), jnp.bfloat16),
            pltpu.VMEM((_CHUNK, _D), jnp.bfloat16),
            pltpu.VMEM((_CHUNK, _D), jnp.float32),
            pltpu.VMEM((_CHUNK, _D), jnp.float32),
            pltpu.VMEM((_ZROWS, _DEG_W), jnp.float32),
            pltpu.VMEM((_CHUNK, _DEG_W), jnp.float32),
            pltpu.SemaphoreType.DMA,
            pltpu.SemaphoreType.DMA,
            pltpu.SemaphoreType.DMA,
            pltpu.SemaphoreType.DMA,
            pltpu.SemaphoreType.DMA,
            pltpu.SemaphoreType.DMA,
        ],
    )
    def sc_kernel(h_hbm, ei_hbm, acc_out, deg_out,
                  acc_sp, deg_sp, idx0, idx1, rbf0, rbf1, rf0, rf1, zdeg, ones,
                  gs0, gs1, ss0, ss1, ds0, ds1):
        c = lax.axis_index("c")
        s = lax.axis_index("s")
        idx = (idx0, idx1)
        rbf = (rbf0, rbf1)
        rf = (rf0, rf1)
        gs, ss, ds = (gs0, gs1), (ss0, ss1), (ds0, ds1)

        zf = jnp.zeros((16,), jnp.float32)
        onesv = jnp.ones((16,), jnp.float32)

        def zbody(i, _):
            for j in range(_D // 16):
                rf0[i, pl.ds(j * 16, 16)] = zf
            zdeg[i, :] = zf
            ones[i, :] = onesv
            return 0

        lax.fori_loop(0, _ZROWS, zbody, 0)

        for k in range(_ROWS_PER_TILE // _ZROWS):
            off = s * _ROWS_PER_TILE + k * _ZROWS
            pltpu.sync_copy(rf0, acc_sp.at[pl.ds(off, _ZROWS)])
            pltpu.sync_copy(zdeg, deg_sp.at[pl.ds(off, _ZROWS)])
        plsc.subcore_barrier()

        tile_base = c * _E_PER_CORE + s * _E_PER_TILE

        def gather_wait(k):
            pltpu.make_async_copy(h_hbm.at[idx[k].at[0]], rbf[k], gs[k]).wait()

        def convert(k):
            def cbody(r, _):
                for t in range(_D // 32):
                    g = rbf[k][r, pl.ds(32 * t, 32)]
                    a, b = plsc.unpack(g, format=plsc.PackFormat.INTERLEAVED,
                                       preferred_element_type=jnp.float32)
                    rf[k][r, pl.ds(32 * t, 16)] = a
                    rf[k][r, pl.ds(32 * t + 16, 16)] = b
                return 0

            lax.fori_loop(0, _CHUNK, cbody, 0)

        def scatter_start(k):
            pltpu.async_copy(rf[k], acc_sp.at[idx[k].at[1]], ss[k], add=True)
            pltpu.async_copy(ones, deg_sp.at[idx[k].at[1]], ds[k], add=True)

        def scatter_wait(k):
            pltpu.make_async_copy(rf[k], acc_sp.at[idx[k].at[1]], ss[k]).wait()
            pltpu.make_async_copy(ones, deg_sp.at[idx[k].at[1]], ds[k]).wait()

        def step(i, cur, prv):
            @pl.when(i >= 1)
            def _():
                gather_wait(prv)
                convert(prv)
                scatter_start(prv)

            @pl.when(i >= 2)
            def _():
                scatter_wait(cur)

            base = tile_base + i * _CHUNK
            pltpu.sync_copy(ei_hbm.at[pl.ds(0, 2), pl.ds(base, _CHUNK)],
                            idx[cur])
            pltpu.async_copy(h_hbm.at[idx[cur].at[0]], rbf[cur], gs[cur])

        def body(i, _):
            @pl.when(i % 2 == 0)
            def _():
                step(i, 0, 1)

            @pl.when(i % 2 == 1)
            def _():
                step(i, 1, 0)
            return 0

        lax.fori_loop(0, _NPT, body, 0)

        last = (_NPT - 1) % 2
        scatter_wait(1 - last)
        gather_wait(last)
        convert(last)
        scatter_start(last)
        scatter_wait(last)
        plsc.subcore_barrier()

        off = s * _ROWS_PER_TILE
        pltpu.sync_copy(acc_sp.at[pl.ds(off, _ROWS_PER_TILE)],
                        acc_out.at[c, pl.ds(off, _ROWS_PER_TILE)])
        pltpu.sync_copy(deg_sp.at[pl.ds(off, _ROWS_PER_TILE)],
                        deg_out.at[c, pl.ds(off, _ROWS_PER_TILE)])

    return sc_kernel(h_tan, edge_index)


def _phase_c_body(a0_ref, a1_ref, d0_ref, d1_ref, o_ref):
    agg = a0_ref[0] + a1_ref[0]
    deg = d0_ref[0][:, :1] + d1_ref[0][:, :1]
    agg = agg / jnp.maximum(deg, 1.0)
    out = _proj(_expmap0(agg))
    xt = jnp.maximum(_logmap0(out), 0.0)
    o_ref[...] = _proj(_expmap0(xt))


def _phase_c(acc, deg):
    blk = 1000
    return pl.pallas_call(
        _phase_c_body,
        grid=(_N // blk,),
        in_specs=[
            pl.BlockSpec((1, blk, _D), lambda i: (0, i, 0)),
            pl.BlockSpec((1, blk, _D), lambda i: (1, i, 0)),
            pl.BlockSpec((1, blk, _DEG_W), lambda i: (0, i, 0)),
            pl.BlockSpec((1, blk, _DEG_W), lambda i: (1, i, 0)),
        ],
        out_specs=pl.BlockSpec((blk, _D), lambda i: (i, 0)),
        out_shape=jax.ShapeDtypeStruct((_N, _D), jnp.float32),
    )(acc, acc, deg, deg)


def kernel(x, edge_index, W, b):
    h_tan = _phase_a(x, W, b.reshape(1, -1))
    acc, deg = _sc_agg(h_tan, edge_index)
    return _phase_c(acc, deg)


# speedup vs baseline: 2.1912x; 2.1912x over previous
import functools

import jax
import jax.numpy as jnp
from jax import lax
from jax.experimental import pallas as pl
from jax.experimental.pallas import tpu as pltpu
from jax.experimental.pallas import tpu_sc as plsc

_MIN_NORM = 1e-15
_BALL_EPS = 4e-3
_N, _E, _D = 10000, 320000, 128

_NC, _NS = 2, 16
_N_PAD = 10240
_ROWS_PER_TILE = _N_PAD // _NS
_CHUNK = 80
_E_PER_CORE = _E // _NC
_E_PER_TILE = _E_PER_CORE // _NS
_NPT = _E_PER_TILE // _CHUNK
_ZROWS = 80
_DEG_W = 16


def _rnorm(v):
    return jnp.maximum(jnp.sqrt(jnp.sum(v * v, axis=-1, keepdims=True)), _MIN_NORM)


def _artanh(v):
    v = jnp.clip(v, -1.0 + 1e-7, 1.0 - 1e-7)
    return 0.5 * (jnp.log1p(v) - jnp.log1p(-v))


def _proj(v):
    n = _rnorm(v)
    maxnorm = 1.0 - _BALL_EPS
    return jnp.where(n > maxnorm, v / n * maxnorm, v)


def _expmap0(v):
    n = _rnorm(v)
    return jnp.tanh(n) * v / n


def _logmap0(v):
    n = _rnorm(v)
    return v / n * _artanh(n)


def _phase_a_body(x_ref, w_ref, b_ref, o_ref):
    x = x_ref[...]
    w = w_ref[...]
    b = b_ref[...]

    x_hyp = _proj(_expmap0(x))

    xn = _rnorm(x_hyp)
    mx = lax.dot_general(x_hyp, w, (((1,), (1,)), ((), ())),
                         preferred_element_type=jnp.float32,
                         precision=lax.Precision.HIGHEST)
    mxn = _rnorm(mx)
    res = jnp.tanh(mxn / xn * _artanh(xn)) * mx / mxn
    res = jnp.where(jnp.all(mx == 0.0, axis=-1, keepdims=True), 0.0, res)
    res = _proj(res)

    hb = _proj(_expmap0(b))

    x2 = jnp.sum(res * res, axis=-1, keepdims=True)
    y2 = jnp.sum(hb * hb, axis=-1, keepdims=True)
    xy = jnp.sum(res * hb, axis=-1, keepdims=True)
    num = (1.0 + 2.0 * xy + y2) * res + (1.0 - x2) * hb
    den = jnp.maximum(1.0 + 2.0 * xy + x2 * y2, _MIN_NORM)
    ma = _proj(num / den)

    o_ref[...] = _logmap0(ma).astype(jnp.bfloat16)


def _phase_a(x, W, b2):
    blk = 1000
    return pl.pallas_call(
        _phase_a_body,
        grid=(_N // blk,),
        in_specs=[
            pl.BlockSpec((blk, _D), lambda i: (i, 0)),
            pl.BlockSpec((_D, _D), lambda i: (0, 0)),
            pl.BlockSpec((1, _D), lambda i: (0, 0)),
        ],
        out_specs=pl.BlockSpec((blk, _D), lambda i: (i, 0)),
        out_shape=jax.ShapeDtypeStruct((_N, _D), jnp.bfloat16),
    )(x, W, b2)


def _sc_agg(h_tan, edge_index):
    @functools.partial(
        pl.kernel,
        out_type=[
            jax.ShapeDtypeStruct((_NC, _N_PAD, _D), jnp.bfloat16),
            jax.ShapeDtypeStruct((_NC, _N_PAD, _DEG_W), jnp.float32),
        ],
        mesh=plsc.VectorSubcoreMesh(core_axis_name="c", subcore_axis_name="s"),
        compiler_params=pltpu.CompilerParams(use_tc_tiling_on_sc=False),
        scratch_types=[
            pltpu.VMEM_SHARED((_N_PAD, _D), jnp.bfloat16),
            pltpu.VMEM_SHARED((_N_PAD, _DEG_W), jnp.float32),
            pltpu.VMEM((2, _CHUNK), jnp.int32),
            pltpu.VMEM((2, _CHUNK), jnp.int32),
            pltpu.VMEM((_CHUNK, _D), jnp.bfloat16),
            pltpu.VMEM((_CHUNK, _D), jnp.bfloat16),
            pltpu.VMEM((_ZROWS, _DEG_W), jnp.float32),
            pltpu.VMEM((_CHUNK, _DEG_W), jnp.float32),
            pltpu.SemaphoreType.DMA,
            pltpu.SemaphoreType.DMA,
            pltpu.SemaphoreType.DMA,
            pltpu.SemaphoreType.DMA,
            pltpu.SemaphoreType.DMA,
            pltpu.SemaphoreType.DMA,
        ],
    )
    def sc_kernel(h_hbm, ei_hbm, acc_out, deg_out,
                  acc_sp, deg_sp, idx0, idx1, rows0, rows1, zdeg, ones,
                  gs0, gs1, ss0, ss1, ds0, ds1):
        c = lax.axis_index("c")
        s = lax.axis_index("s")
        idx = (idx0, idx1)
        rows = (rows0, rows1)
        gs, ss, ds = (gs0, gs1), (ss0, ss1), (ds0, ds1)

        zfb = jnp.zeros((32,), jnp.bfloat16)
        zf = jnp.zeros((16,), jnp.float32)
        onesv = jnp.ones((16,), jnp.float32)

        def zbody(i, _):
            for j in range(_D // 32):
                rows0[i, pl.ds(j * 32, 32)] = zfb
            zdeg[i, :] = zf
            ones[i, :] = onesv
            return 0

        lax.fori_loop(0, _ZROWS, zbody, 0)

        for k in range(_ROWS_PER_TILE // _ZROWS):
            off = s * _ROWS_PER_TILE + k * _ZROWS
            pltpu.sync_copy(rows0, acc_sp.at[pl.ds(off, _ZROWS)])
            pltpu.sync_copy(zdeg, deg_sp.at[pl.ds(off, _ZROWS)])
        plsc.subcore_barrier()

        tile_base = c * _E_PER_CORE + s * _E_PER_TILE

        def gather_wait(k):
            pltpu.make_async_copy(h_hbm.at[idx[k].at[0]], rows[k], gs[k]).wait()

        def scatter_start(k):
            pltpu.async_copy(rows[k], acc_sp.at[idx[k].at[1]], ss[k], add=True)
            pltpu.async_copy(ones, deg_sp.at[idx[k].at[1]], ds[k], add=True)

        def scatter_wait(k):
            pltpu.make_async_copy(rows[k], acc_sp.at[idx[k].at[1]], ss[k]).wait()
            pltpu.make_async_copy(ones, deg_sp.at[idx[k].at[1]], ds[k]).wait()

        def step(i, cur, prv):
            @pl.when(i >= 1)
            def _():
                gather_wait(prv)
                scatter_start(prv)

            @pl.when(i >= 2)
            def _():
                scatter_wait(cur)

            base = tile_base + i * _CHUNK
            pltpu.sync_copy(ei_hbm.at[pl.ds(0, 2), pl.ds(base, _CHUNK)],
                            idx[cur])
            pltpu.async_copy(h_hbm.at[idx[cur].at[0]], rows[cur], gs[cur])

        def body(i, _):
            @pl.when(i % 2 == 0)
            def _():
                step(i, 0, 1)

            @pl.when(i % 2 == 1)
            def _():
                step(i, 1, 0)
            return 0

        lax.fori_loop(0, _NPT, body, 0)

        last = (_NPT - 1) % 2
        scatter_wait(1 - last)
        gather_wait(last)
        scatter_start(last)
        scatter_wait(last)
        plsc.subcore_barrier()

        off = s * _ROWS_PER_TILE
        pltpu.sync_copy(acc_sp.at[pl.ds(off, _ROWS_PER_TILE)],
                        acc_out.at[c, pl.ds(off, _ROWS_PER_TILE)])
        pltpu.sync_copy(deg_sp.at[pl.ds(off, _ROWS_PER_TILE)],
                        deg_out.at[c, pl.ds(off, _ROWS_PER_TILE)])

    return sc_kernel(h_tan, edge_index)


def _phase_c_body(a0_ref, a1_ref, d0_ref, d1_ref, o_ref):
    agg = a0_ref[0].astype(jnp.float32) + a1_ref[0].astype(jnp.float32)
    deg = d0_ref[0][:, :1] + d1_ref[0][:, :1]
    agg = agg / jnp.maximum(deg, 1.0)
    out = _proj(_expmap0(agg))
    xt = jnp.maximum(_logmap0(out), 0.0)
    o_ref[...] = _proj(_expmap0(xt))


def _phase_c(acc, deg):
    blk = 1000
    return pl.pallas_call(
        _phase_c_body,
        grid=(_N // blk,),
        in_specs=[
            pl.BlockSpec((1, blk, _D), lambda i: (0, i, 0)),
            pl.BlockSpec((1, blk, _D), lambda i: (1, i, 0)),
            pl.BlockSpec((1, blk, _DEG_W), lambda i: (0, i, 0)),
            pl.BlockSpec((1, blk, _DEG_W), lambda i: (1, i, 0)),
        ],
        out_specs=pl.BlockSpec((blk, _D), lambda i: (i, 0)),
        out_shape=jax.ShapeDtypeStruct((_N, _D), jnp.float32),
    )(acc, acc, deg, deg)


def kernel(x, edge_index, W, b):
    h_tan = _phase_a(x, W, b.reshape(1, -1))
    acc, deg = _sc_agg(h_tan, edge_index)
    return _phase_c(acc, deg)
